# baseline (speedup 1.0000x reference)
import functools

import jax
import jax.numpy as jnp
from jax import lax
from jax.experimental import pallas as pl
from jax.experimental.pallas import tpu as pltpu
from jax.experimental.pallas import tpu_sc as plsc

B = 1024
L = 50
H = 8
N = 10000
LANES = 16
NC = 2
NS = 16
NW = NC * NS
BPW = B // NW
LPAD = 64
IDXN = L * LPAD
OUTB = L * L * H
SEQW = BPW * L
NCHUNK = IDXN // 128


def _sc_body(seq_hbm, table_hbm, out_hbm, seq_v, idx_v, vals_v, outb, sem):
    wid = lax.axis_index("s") * NC + lax.axis_index("c")
    base_b = wid * BPW
    seq_v[pl.ds(SEQW, LANES)] = jnp.zeros((LANES,), jnp.int32)
    pltpu.sync_copy(seq_hbm.at[pl.ds(base_b * L, SEQW)], seq_v.at[pl.ds(0, SEQW)])

    iota = lax.iota(jnp.int32, LANES)
    iota8 = iota * H
    tail_mask = iota < (L - 48)

    def batch_body(t, carry):
        sbase = t * L

        def row_body(i, c2):
            r = seq_v[pl.ds(sbase + i, LANES)][0] * N
            for c in range(4):
                sv = seq_v[pl.ds(sbase + c * LANES, LANES)]
                idx_v[pl.ds(i * LPAD + c * LANES, LANES)] = sv + r
            return c2

        lax.fori_loop(0, L, row_body, 0)

        def g_body(k, c2):
            pltpu.async_copy(
                table_hbm.at[idx_v.at[pl.ds(k * 128, 128)]],
                vals_v.at[pl.ds(k * 128, 128)],
                sem,
            ).wait()
            return c2

        lax.fori_loop(0, NCHUNK, g_body, 0)

        def b_body(i, c2):
            for c in range(4):
                v = vals_v[pl.ds(i * LPAD + c * LANES, LANES)]
                ivec = iota8 + (i * L * H + c * LANES * H)
                for h in range(H):
                    if c < 3:
                        plsc.store_scatter(outb, [ivec + h], v)
                    else:
                        plsc.store_scatter(outb, [ivec + h], v, mask=tail_mask)
            return c2

        lax.fori_loop(0, L, b_body, 0)
        pltpu.sync_copy(outb, out_hbm.at[pl.ds((base_b + t) * OUTB, OUTB)])
        return carry

    lax.fori_loop(0, BPW, batch_body, 0)


@jax.jit
def _sc_call(seq_flat, table_flat):
    return pl.kernel(
        _sc_body,
        out_type=jax.ShapeDtypeStruct((B * OUTB,), jnp.int32),
        mesh=plsc.VectorSubcoreMesh(
            core_axis_name="c", subcore_axis_name="s",
            num_cores=NC, num_subcores=NS,
        ),
        scratch_types=[
            pltpu.VMEM((SEQW + LANES,), jnp.int32),
            pltpu.VMEM((IDXN,), jnp.int32),
            pltpu.VMEM((IDXN,), jnp.int32),
            pltpu.VMEM((OUTB,), jnp.int32),
            pltpu.SemaphoreType.DMA,
        ],
        compiler_params=pltpu.CompilerParams(needs_layout_passes=False),
    )(seq_flat, table_flat)


def kernel(user_seq, spatial_pos_table):
    out = _sc_call(user_seq.reshape(-1), spatial_pos_table.reshape(-1))
    return out.reshape(B, L, L, H)


# speedup vs baseline: 1.2911x; 1.2911x over previous
import jax
import jax.numpy as jnp
from jax import lax
from jax.experimental import pallas as pl
from jax.experimental.pallas import tpu as pltpu
from jax.experimental.pallas import tpu_sc as plsc

B = 1024
L = 50
H = 8
N = 10000
LANES = 16
NC = 2
NS = 16
NW = NC * NS
BPW = B // NW
LPAD = 64
IDXN = L * LPAD
OUTB = L * L * H
SEQW = BPW * L
NCHUNK = IDXN // 128


def _sc_body(seq_hbm, table_hbm, out_hbm,
             seq_v, idx0, idx1, vals0, vals1, outb0, outb1, gsem, wsem):
    wid = lax.axis_index("s") * NC + lax.axis_index("c")
    base_b = wid * BPW
    seq_v[pl.ds(SEQW, LANES)] = jnp.zeros((LANES,), jnp.int32)
    pltpu.sync_copy(seq_hbm.at[pl.ds(base_b * L, SEQW)], seq_v.at[pl.ds(0, SEQW)])

    iota = lax.iota(jnp.int32, LANES)
    iota8 = iota * H
    tail_mask = iota < (L - 48)
    slots = ((idx0, vals0, outb0), (idx1, vals1, outb1))

    def build_and_fire(t, idx_v, vals_v):
        sbase = t * L
        svs = [seq_v[pl.ds(sbase + c * LANES, LANES)] for c in range(4)]

        def row_body(i, c2):
            r = seq_v[pl.ds(sbase + i, LANES)][0] * N
            for c in range(4):
                idx_v[pl.ds(i * LPAD + c * LANES, LANES)] = svs[c] + r
            return c2

        lax.fori_loop(0, L, row_body, 0)

        def g_body(k, c2):
            pltpu.async_copy(
                table_hbm.at[idx_v.at[pl.ds(k * 128, 128)]],
                vals_v.at[pl.ds(k * 128, 128)],
                gsem,
            )
            return c2

        lax.fori_loop(0, NCHUNK, g_body, 0)

    def drain_gather(vals_v):
        pltpu.make_async_copy(out_hbm.at[pl.ds(0, IDXN)], vals_v, gsem).wait()

    def drain_write(outb_v):
        pltpu.make_async_copy(outb_v, out_hbm.at[pl.ds(0, OUTB)], wsem).wait()

    def broadcast(vals_v, outb_v):
        def b_body(i, c2):
            for c in range(4):
                v = vals_v[pl.ds(i * LPAD + c * LANES, LANES)]
                ivec = iota8 + (i * L * H + c * LANES * H)
                for h in range(H):
                    if c < 3:
                        plsc.store_scatter(outb_v, [ivec + h], v)
                    else:
                        plsc.store_scatter(outb_v, [ivec + h], v, mask=tail_mask)
            return c2

        lax.fori_loop(0, L, b_body, 0)

    for s in range(2):
        build_and_fire(s, slots[s][0], slots[s][1])

    def pair_body(p, c2):
        for s in range(2):
            idx_v, vals_v, outb_v = slots[s]
            t = p * 2 + s
            drain_gather(vals_v)

            @pl.when(p > 0)
            def _():
                drain_write(outb_v)

            broadcast(vals_v, outb_v)
            pltpu.async_copy(
                outb_v, out_hbm.at[pl.ds((base_b + t) * OUTB, OUTB)], wsem)

            @pl.when(t + 2 < BPW)
            def _():
                build_and_fire(t + 2, idx_v, vals_v)

        return c2

    lax.fori_loop(0, BPW // 2, pair_body, 0)
    drain_write(outb0)
    drain_write(outb1)


@jax.jit
def _sc_call(seq_flat, table_flat):
    return pl.kernel(
        _sc_body,
        out_type=jax.ShapeDtypeStruct((B * OUTB,), jnp.int32),
        mesh=plsc.VectorSubcoreMesh(
            core_axis_name="c", subcore_axis_name="s",
            num_cores=NC, num_subcores=NS,
        ),
        scratch_types=[
            pltpu.VMEM((SEQW + LANES,), jnp.int32),
            pltpu.VMEM((IDXN,), jnp.int32),
            pltpu.VMEM((IDXN,), jnp.int32),
            pltpu.VMEM((IDXN,), jnp.int32),
            pltpu.VMEM((IDXN,), jnp.int32),
            pltpu.VMEM((OUTB,), jnp.int32),
            pltpu.VMEM((OUTB,), jnp.int32),
            pltpu.SemaphoreType.DMA,
            pltpu.SemaphoreType.DMA,
        ],
        compiler_params=pltpu.CompilerParams(needs_layout_passes=False),
    )(seq_flat, table_flat)


def kernel(user_seq, spatial_pos_table):
    out = _sc_call(user_seq.reshape(-1), spatial_pos_table.reshape(-1))
    return out.reshape(B, L, L, H)


# speedup vs baseline: 4.5823x; 3.5493x over previous
import jax
import jax.numpy as jnp
from jax import lax
from jax.experimental import pallas as pl
from jax.experimental.pallas import tpu as pltpu
from jax.experimental.pallas import tpu_sc as plsc

B = 1024
L = 50
H = 8
N = 10000
LANES = 16
NC = 2
NS = 16
NW = NC * NS
BBLK = B // (NW // 4)
IJTOT = L * L
IJW = IJTOT // 4
CH = 25
NCHUNKS = IJW // CH


def _sc_body(seq_hbm, table_hbm, out_hbm,
             seq_blk, ro_v, co_v, idx0, idx1, vals0, vals1, slab0, slab1,
             gsem, wsem):
    wid = lax.axis_index("s") * NC + lax.axis_index("c")
    g = wid >> 2
    q = wid & 3
    ij_base = q * IJW

    pltpu.sync_copy(seq_hbm.at[pl.ds(g * BBLK * L, BBLK * L)], seq_blk)

    iota = lax.iota(jnp.int32, LANES)
    gidx_base = iota * L

    def t_body(i, carry):
        for c in range(8):
            v = plsc.load_gather(seq_blk, [gidx_base + (c * 16 * L + i)])
            ro_v[pl.ds(i * BBLK + c * 16, 16)] = v * N
            co_v[pl.ds(i * BBLK + c * 16, 16)] = v
        return carry

    lax.fori_loop(0, L, t_body, 0)

    slots = ((idx0, vals0, slab0), (idx1, vals1, slab1))

    def build_and_fire(n, idx_v, vals_v):
        def ij_body(m, carry):
            ij = ij_base + n * CH + m
            i = ij // L
            j = ij - i * L
            for c in range(8):
                rv = ro_v[pl.ds(i * BBLK + c * 16, 16)]
                cv = co_v[pl.ds(j * BBLK + c * 16, 16)]
                idx_v[m, pl.ds(c * 16, 16)] = rv + cv
            pltpu.async_copy(
                table_hbm.at[idx_v.at[m]], vals_v.at[m], gsem)
            return carry

        lax.fori_loop(0, CH, ij_body, 0)

    def drain_gather(vals_v):
        pltpu.make_async_copy(out_hbm.at[pl.ds(0, CH), 0, 0], vals_v, gsem).wait()

    def drain_write(slab_v):
        pltpu.make_async_copy(slab_v, out_hbm.at[pl.ds(0, CH), 0], wsem).wait()

    def broadcast(vals_v, slab_v):
        def b_body(m, carry):
            for c in range(8):
                v = vals_v[m, pl.ds(c * 16, 16)]
                for h in range(H):
                    slab_v[m, h, pl.ds(c * 16, 16)] = v
            return carry

        lax.fori_loop(0, CH, b_body, 0)

    for s in range(2):
        build_and_fire(s, slots[s][0], slots[s][1])

    def chunk_body(p, carry):
        for s in range(2):
            idx_v, vals_v, slab_v = slots[s]
            n = p * 2 + s

            @pl.when(n < NCHUNKS)
            def _():
                drain_gather(vals_v)

                @pl.when(p > 0)
                def _():
                    drain_write(slab_v)

                broadcast(vals_v, slab_v)
                pltpu.async_copy(
                    slab_v, out_hbm.at[pl.ds(ij_base + n * CH, CH), g], wsem)

                @pl.when(n + 2 < NCHUNKS)
                def _():
                    build_and_fire(n + 2, idx_v, vals_v)

        return carry

    lax.fori_loop(0, (NCHUNKS + 1) // 2, chunk_body, 0)
    drain_write(slab0)
    drain_write(slab1)


@jax.jit
def _sc_call(seq_flat, table_flat):
    return pl.kernel(
        _sc_body,
        out_type=jax.ShapeDtypeStruct((IJTOT, B // BBLK, H, BBLK), jnp.int32),
        mesh=plsc.VectorSubcoreMesh(
            core_axis_name="c", subcore_axis_name="s",
            num_cores=NC, num_subcores=NS,
        ),
        scratch_types=[
            pltpu.VMEM((BBLK * L,), jnp.int32),
            pltpu.VMEM((L * BBLK,), jnp.int32),
            pltpu.VMEM((L * BBLK,), jnp.int32),
            pltpu.VMEM((CH, BBLK), jnp.int32),
            pltpu.VMEM((CH, BBLK), jnp.int32),
            pltpu.VMEM((CH, BBLK), jnp.int32),
            pltpu.VMEM((CH, BBLK), jnp.int32),
            pltpu.VMEM((CH, H, BBLK), jnp.int32),
            pltpu.VMEM((CH, H, BBLK), jnp.int32),
            pltpu.SemaphoreType.DMA,
            pltpu.SemaphoreType.DMA,
        ],
        compiler_params=pltpu.CompilerParams(needs_layout_passes=False),
    )(seq_flat, table_flat)


def kernel(user_seq, spatial_pos_table):
    out4 = _sc_call(user_seq.reshape(-1), spatial_pos_table.reshape(-1))
    return out4.transpose(1, 3, 0, 2).reshape(B, L, L, H)
